# Initial kernel scaffold; baseline (speedup 1.0000x reference)
import functools

import jax
import jax.numpy as jnp
from jax import lax
from jax.experimental import pallas as pl
from jax.experimental.pallas import tpu as pltpu
from jax.experimental.pallas import tpu_sc as plsc

NUM_BUCKETS = 32
N_HEADS = 16
SEQ = 2048
STRIP = 2 * SEQ

NUM_CORES = 2
NUM_SUBCORES = 16
LANES = 16

ROWS_PER_WORKER = SEQ // NUM_CORES
FIRE = 16


def _make_fill():
    mesh = plsc.VectorSubcoreMesh(core_axis_name="c", subcore_axis_name="s")

    @functools.partial(
        pl.kernel,
        mesh=mesh,
        out_type=jax.ShapeDtypeStruct((N_HEADS * SEQ * SEQ,), jnp.float32),
        scratch_types=[
            pltpu.VMEM((NUM_BUCKETS, N_HEADS), jnp.float32),
            pltpu.VMEM((STRIP,), jnp.float32),
            pltpu.SemaphoreType.DMA,
        ],
    )
    def fill(table_hbm, out_hbm, table_v, strip_v, sem):
        head = lax.axis_index("s")
        half = lax.axis_index("c")

        pltpu.sync_copy(table_hbm, table_v)

        head_idx = jnp.full((LANES,), head, dtype=jnp.int32)
        lane = lax.broadcasted_iota(jnp.int32, (LANES,), 0)

        def strip_body(j, carry):
            doff = j * LANES + lane
            d = doff - (SEQ - 1)
            n = jnp.minimum(jnp.abs(d), 15)
            b = n + 16 * (d > 0).astype(jnp.int32)
            vals = plsc.load_gather(table_v, [b, head_idx])
            strip_v[pl.ds(j * LANES, LANES)] = vals
            return carry

        lax.fori_loop(0, STRIP // LANES, strip_body, 0)

        q_base = half * ROWS_PER_WORKER
        row_base = head * SEQ + q_base

        def dma_body(g, carry):
            q0 = q_base + g * FIRE
            r0 = row_base + g * FIRE
            copies = []
            for b in range(FIRE):
                src = strip_v.at[pl.ds((SEQ - 1) - (q0 + b), SEQ)]
                dst = out_hbm.at[pl.ds((r0 + b) * SEQ, SEQ)]
                copies.append(pltpu.async_copy(src, dst, sem))
            for c in copies:
                c.wait()
            return carry

        lax.fori_loop(0, ROWS_PER_WORKER // FIRE, dma_body, 0)

    return fill


_fill = _make_fill()


def kernel(relative_attention_bias, qlen, klen):
    del qlen, klen
    flat = _fill(relative_attention_bias)
    return flat.reshape(1, N_HEADS, SEQ, SEQ)


# speedup vs baseline: 42.6709x; 42.6709x over previous
import functools

import jax
import jax.numpy as jnp
from jax import lax
from jax.experimental import pallas as pl
from jax.experimental.pallas import tpu as pltpu
from jax.experimental.pallas import tpu_sc as plsc

NUM_BUCKETS = 32
N_HEADS = 16
SEQ = 2048
STRIP = 2 * SEQ

NUM_CORES = 2
NUM_SUBCORES = 16
LANES = 16

ROWS_PER_WORKER = SEQ // NUM_CORES
FIRE = 16


def _make_fill():
    mesh = plsc.VectorSubcoreMesh(core_axis_name="c", subcore_axis_name="s")

    @functools.partial(
        pl.kernel,
        mesh=mesh,
        out_type=jax.ShapeDtypeStruct((N_HEADS * SEQ * SEQ,), jnp.float32),
        scratch_types=[
            pltpu.VMEM((NUM_BUCKETS, N_HEADS), jnp.float32),
            pltpu.VMEM((8 * STRIP,), jnp.float32),
            pltpu.SemaphoreType.DMA,
        ],
        compiler_params=pltpu.CompilerParams(needs_layout_passes=False),
    )
    def fill(table_hbm, out_hbm, table_v, strips_v, sem):
        head = lax.axis_index("s")
        half = lax.axis_index("c")

        pltpu.sync_copy(table_hbm, table_v)

        head_idx = jnp.full((LANES,), head, dtype=jnp.int32)
        lane = lax.broadcasted_iota(jnp.int32, (LANES,), 0)

        for r in range(8):

            def strip_body(j, carry, r=r):
                doff = j * LANES + lane + r
                d = doff - (SEQ - 1)
                n = jnp.minimum(jnp.abs(d), 15)
                b = n + 16 * (d > 0).astype(jnp.int32)
                vals = plsc.load_gather(table_v, [b, head_idx])
                strips_v[pl.ds(r * STRIP + j * LANES, LANES)] = vals
                return carry

            lax.fori_loop(0, STRIP // LANES, strip_body, 0)

        q_base = half * ROWS_PER_WORKER
        row_base = head * SEQ + q_base

        def dma_body(g, carry):
            q0 = q_base + g * FIRE
            r0 = row_base + g * FIRE
            copies = []
            for b in range(FIRE):
                r = (7 - b) % 8
                base = (SEQ - 8 - (b - b % 8)) - q0
                src = strips_v.at[pl.ds(r * STRIP + base, SEQ)]
                dst = out_hbm.at[pl.ds((r0 + b) * SEQ, SEQ)]
                copies.append(pltpu.async_copy(src, dst, sem))
            for c in copies:
                c.wait()
            return carry

        lax.fori_loop(0, ROWS_PER_WORKER // FIRE, dma_body, 0)

    return fill


_fill = _make_fill()


def kernel(relative_attention_bias, qlen, klen):
    del qlen, klen
    flat = _fill(relative_attention_bias)
    return flat.reshape(1, N_HEADS, SEQ, SEQ)
